# baseline (speedup 1.0000x reference)
import numpy as np
import jax
import jax.numpy as jnp
from jax.experimental import pallas as pl
from jax.experimental.pallas import tpu as pltpu

ALPHA = 0.98
DELTA = 2.0
S = 0.025
A = 1.0 - S
EPS = 1e-6
SQRT_DELTA = DELTA ** 0.5

L = 256
M_TILE = 256
T_BLK = 4096


def _coeff_matrix() -> np.ndarray:
    i = np.arange(L, dtype=np.float64)[:, None]
    k = np.arange(L, dtype=np.float64)[None, :]
    d = k - i
    C = np.where(d >= 0, S * np.power(A, np.maximum(d, 0.0)), 0.0)
    return C.astype(np.float32)


def _pow_row() -> np.ndarray:
    return (A ** (np.arange(L, dtype=np.float64) + 1.0)).astype(np.float32)[None, :]


def _pcen_body(c_ref, p_ref, x_ref, o_ref, carry_ref):
    t = pl.program_id(1)

    @pl.when(t == 0)
    def _():
        carry_ref[...] = x_ref[:, 0:1]

    carry = carry_ref[...]
    C = c_ref[...]
    powrow = p_ref[...]
    for j in range(T_BLK // L):
        xc = x_ref[:, j * L:(j + 1) * L]
        local = jax.lax.dot_general(
            xc, C, (((1,), (0,)), ((), ())),
            preferred_element_type=jnp.float32)
        sm = local + carry * powrow
        carry = sm[:, L - 1:L]
        v = xc * jnp.exp2(-ALPHA * jnp.log2(sm + EPS)) + DELTA
        o_ref[:, j * L:(j + 1) * L] = v * jax.lax.rsqrt(v) - SQRT_DELTA
    carry_ref[...] = carry


def kernel(x):
    B, Cdim, T = x.shape
    M = B * Cdim
    xf = x.reshape(M, T)
    Cm = jnp.asarray(_coeff_matrix())
    pr = jnp.asarray(_pow_row())
    out = pl.pallas_call(
        _pcen_body,
        out_shape=jax.ShapeDtypeStruct((M, T), jnp.float32),
        grid=(M // M_TILE, T // T_BLK),
        in_specs=[
            pl.BlockSpec((L, L), lambda m, t: (0, 0)),
            pl.BlockSpec((1, L), lambda m, t: (0, 0)),
            pl.BlockSpec((M_TILE, T_BLK), lambda m, t: (m, t)),
        ],
        out_specs=pl.BlockSpec((M_TILE, T_BLK), lambda m, t: (m, t)),
        scratch_shapes=[pltpu.VMEM((M_TILE, 1), jnp.float32)],
        compiler_params=pltpu.CompilerParams(
            dimension_semantics=("parallel", "arbitrary"),
            vmem_limit_bytes=50 * 1024 * 1024,
        ),
        name="pcen",
    )(Cm, pr, xf)
    return out.reshape(B, Cdim, T)


# speedup vs baseline: 1.0905x; 1.0905x over previous
import numpy as np
import jax
import jax.numpy as jnp
from jax.experimental import pallas as pl
from jax.experimental.pallas import tpu as pltpu

ALPHA = 0.98
DELTA = 2.0
S = 0.025
A = 1.0 - S
EPS = 1e-6
SQRT_DELTA = DELTA ** 0.5

L = 256
M_TILE = 512
T_BLK = 4096


def _coeff_matrix() -> np.ndarray:
    i = np.arange(L, dtype=np.float64)[:, None]
    k = np.arange(L, dtype=np.float64)[None, :]
    d = k - i
    C = np.where(d >= 0, S * np.power(A, np.maximum(d, 0.0)), 0.0)
    return C.astype(np.float32)


def _pow_row() -> np.ndarray:
    return (A ** (np.arange(L, dtype=np.float64) + 1.0)).astype(np.float32)[None, :]


def _pcen_body(c_ref, p_ref, x_ref, o_ref, carry_ref):
    t = pl.program_id(1)

    @pl.when(t == 0)
    def _():
        carry_ref[...] = x_ref[:, 0:1]

    carry = carry_ref[...]
    C = c_ref[...]
    powrow = p_ref[...]
    for j in range(T_BLK // L):
        xc = x_ref[:, j * L:(j + 1) * L]
        local = jax.lax.dot_general(
            xc, C, (((1,), (0,)), ((), ())),
            preferred_element_type=jnp.float32)
        sm = local + carry * powrow
        carry = sm[:, L - 1:L]
        v = xc * jnp.exp2(-ALPHA * jnp.log2(sm + EPS)) + DELTA
        o_ref[:, j * L:(j + 1) * L] = v * jax.lax.rsqrt(v) - SQRT_DELTA
    carry_ref[...] = carry


def kernel(x):
    B, Cdim, T = x.shape
    M = B * Cdim
    xf = x.reshape(M, T)
    Cm = jnp.asarray(_coeff_matrix())
    pr = jnp.asarray(_pow_row())
    out = pl.pallas_call(
        _pcen_body,
        out_shape=jax.ShapeDtypeStruct((M, T), jnp.float32),
        grid=(M // M_TILE, T // T_BLK),
        in_specs=[
            pl.BlockSpec((L, L), lambda m, t: (0, 0)),
            pl.BlockSpec((1, L), lambda m, t: (0, 0)),
            pl.BlockSpec((M_TILE, T_BLK), lambda m, t: (m, t)),
        ],
        out_specs=pl.BlockSpec((M_TILE, T_BLK), lambda m, t: (m, t)),
        scratch_shapes=[pltpu.VMEM((M_TILE, 1), jnp.float32)],
        compiler_params=pltpu.CompilerParams(
            dimension_semantics=("parallel", "arbitrary"),
            vmem_limit_bytes=50 * 1024 * 1024,
        ),
        name="pcen",
    )(Cm, pr, xf)
    return out.reshape(B, Cdim, T)
